# Initial kernel scaffold; baseline (speedup 1.0000x reference)
import jax
import jax.numpy as jnp
from jax.experimental import pallas as pl


def kernel(scores):
    raise NotImplementedError("write your pallas kernel here")


# speedup vs baseline: 2.9062x; 2.9062x over previous
import jax
import jax.numpy as jnp
from jax.experimental import pallas as pl
from jax.experimental.pallas import tpu as pltpu

_BLK_ROWS = 256


def _softmax_block(x_ref, o_ref):
    x = x_ref[...]
    m = jnp.max(x, axis=-1, keepdims=True)
    e = jnp.exp(x - m)
    s = jnp.sum(e, axis=-1, keepdims=True)
    o_ref[...] = e * (1.0 / s)


def kernel(scores):
    b, h, q, k = scores.shape
    rows = b * h * q
    x = scores.reshape(rows, k)
    out = pl.pallas_call(
        _softmax_block,
        grid=(rows // _BLK_ROWS,),
        in_specs=[pl.BlockSpec((_BLK_ROWS, k), lambda i: (i, 0))],
        out_specs=pl.BlockSpec((_BLK_ROWS, k), lambda i: (i, 0)),
        out_shape=jax.ShapeDtypeStruct((rows, k), scores.dtype),
        compiler_params=pltpu.CompilerParams(
            dimension_semantics=("arbitrary",),
        ),
    )(x)
    return out.reshape(b, h, q, k)
